# baseline (speedup 1.0000x reference)
import functools

import jax
import jax.numpy as jnp
from jax import lax
from jax.experimental import pallas as pl
from jax.experimental.pallas import tpu as pltpu
from jax.experimental.pallas import tpu_sc as plsc

_VOCAB = 1000000
_EMBED = 64
_B = 4096
_L = 200
_N = _B * _L

_GATHER_WINDOW = 128


def _sc_gather(token_table, flat_ids):
    mesh = plsc.VectorSubcoreMesh(core_axis_name="c", subcore_axis_name="s")

    @functools.partial(
        pl.kernel,
        out_type=jax.ShapeDtypeStruct((_N, _EMBED), jnp.float32),
        mesh=mesh,
        compiler_params=pltpu.CompilerParams(use_tc_tiling_on_sc=False),
    )
    def gather_kernel(table_hbm, idx_hbm, out_hbm):
        def body(i_vmem, o_vmem):
            pltpu.sync_copy(table_hbm.at[i_vmem.at[0]], o_vmem)

        pltpu.emit_pipeline(
            body,
            grid=(_N // _GATHER_WINDOW,),
            in_specs=[
                pl.BlockSpec((1, _GATHER_WINDOW), index_map=lambda i: (0, i))
            ],
            out_specs=[
                pl.BlockSpec((_GATHER_WINDOW, _EMBED), index_map=lambda i: (i, 0))
            ],
            core_axis_name=("c", "s"),
            dimension_semantics=(pltpu.PARALLEL,),
        )(idx_hbm, out_hbm)

    return gather_kernel(token_table, flat_ids.reshape(1, _N))


def _ln_body(tok_ref, pos_ref, gamma_ref, beta_ref, out_ref):
    y = tok_ref[...] + pos_ref[...]
    mean = jnp.mean(y, axis=-1, keepdims=True)
    var = jnp.mean(jnp.square(y - mean), axis=-1, keepdims=True)
    normed = (y - mean) * lax.rsqrt(var + 1e-5)
    out_ref[...] = normed * gamma_ref[...] + beta_ref[...]


_LN_BB = 16


def _tc_layernorm(tok3, pos3, gamma3, beta3):
    return pl.pallas_call(
        _ln_body,
        grid=(_B // _LN_BB,),
        in_specs=[
            pl.BlockSpec((_LN_BB, _L, _EMBED), lambda i: (i, 0, 0)),
            pl.BlockSpec((1, _L, _EMBED), lambda i: (0, 0, 0)),
            pl.BlockSpec((1, 1, _EMBED), lambda i: (0, 0, 0)),
            pl.BlockSpec((1, 1, _EMBED), lambda i: (0, 0, 0)),
        ],
        out_specs=pl.BlockSpec((_LN_BB, _L, _EMBED), lambda i: (i, 0, 0)),
        out_shape=jax.ShapeDtypeStruct((_B, _L, _EMBED), jnp.float32),
    )(tok3, pos3, gamma3, beta3)


def kernel(input_ids, token_table, pos_table, gamma, beta):
    flat_ids = input_ids.reshape(-1).astype(jnp.int32)
    tok = _sc_gather(token_table, flat_ids)
    tok3 = tok.reshape(_B, _L, _EMBED)
    pos3 = pos_table[:_L].reshape(1, _L, _EMBED)
    gamma3 = gamma.reshape(1, 1, _EMBED)
    beta3 = beta.reshape(1, 1, _EMBED)
    return _tc_layernorm(tok3, pos3, gamma3, beta3)
